# baseline (speedup 1.0000x reference)
import functools
import math

import jax
import jax.numpy as jnp
from jax import lax
from jax.experimental import pallas as pl
from jax.experimental.pallas import tpu as pltpu
from jax.experimental.pallas import tpu_sc as plsc

D_MODEL = 64
SCALE = math.sqrt(D_MODEL)

NUM_CORES = 2
NUM_SUBCORES = 16
NUM_WORKERS = NUM_CORES * NUM_SUBCORES

LANES = 16
CHUNK = 128
NBUF = 4


def _make_kernel(b: int, s: int, vocab: int, d: int):
  assert d == LANES * 4 and b % (NUM_WORKERS * CHUNK) == 0
  assert s % NBUF == 0
  n_groups = s // NBUF
  v2 = vocab // 2

  mesh = plsc.VectorSubcoreMesh(core_axis_name="c", subcore_axis_name="s")

  @functools.partial(
      pl.kernel,
      mesh=mesh,
      out_type=jax.ShapeDtypeStruct((s, d, b), jnp.float32),
      scratch_types=[
          pltpu.VMEM((s, CHUNK), jnp.int32),
          [pltpu.VMEM((CHUNK,), jnp.int32) for _ in range(NBUF)],
          [pltpu.VMEM((CHUNK,), jnp.int32) for _ in range(NBUF)],
          [pltpu.VMEM((CHUNK, 2 * d), jnp.float32) for _ in range(NBUF)],
          [pltpu.VMEM((d, CHUNK), jnp.float32) for _ in range(NBUF)],
          [pltpu.SemaphoreType.DMA for _ in range(NBUF)],
          [pltpu.SemaphoreType.DMA for _ in range(NBUF)],
      ],
      compiler_params=pltpu.CompilerParams(
          use_tc_tiling_on_sc=True, needs_layout_passes=False
      ),
  )
  def emb_kernel(
      x_hbm, w_hbm, out_hbm, idx_v, gidx, hoff, gbuf, tbuf, gat_sem, scat_sem
  ):
    wid = lax.axis_index("s") * NUM_CORES + lax.axis_index("c")
    pltpu.sync_copy(x_hbm.at[:, wid], idx_v)

    def gather_start(si, buf):
      def mk_idx(k, c2):
        ids = idx_v[si, pl.ds(k * LANES, LANES)]
        gidx[buf][pl.ds(k * LANES, LANES)] = lax.shift_right_logical(ids, 1)
        hoff[buf][pl.ds(k * LANES, LANES)] = (ids & 1) * d
        return c2

      lax.fori_loop(0, CHUNK // LANES, mk_idx, 0, unroll=4)
      pltpu.make_async_copy(w_hbm.at[gidx[buf]], gbuf[buf], gat_sem[buf]).start()

    def gather_wait(buf):
      pltpu.make_async_copy(w_hbm.at[gidx[buf]], gbuf[buf], gat_sem[buf]).wait()

    def scat_start(si, buf):
      dst = out_hbm.at[si, :, pl.ds(wid * CHUNK, CHUNK)]
      pltpu.make_async_copy(tbuf[buf], dst, scat_sem[buf]).start()

    def scat_wait(si, buf):
      dst = out_hbm.at[si, :, pl.ds(wid * CHUNK, CHUNK)]
      pltpu.make_async_copy(tbuf[buf], dst, scat_sem[buf]).wait()

    def transpose_scale(buf):
      iota = jax.lax.broadcasted_iota(jnp.int32, (LANES,), 0)
      for tg in range(CHUNK // LANES):
        rowi = iota + tg * LANES
        ho = hoff[buf][pl.ds(tg * LANES, LANES)]

        @plsc.parallel_loop(0, d, unroll=16)
        def dloop(dd, rowi=rowi, ho=ho, tg=tg):
          v = plsc.load_gather(gbuf[buf], [rowi, ho + dd])
          tbuf[buf][dd, pl.ds(tg * LANES, LANES)] = v * SCALE

    for buf in range(NBUF):
      gather_start(buf, buf)

    def group_body(g, carry):
      s0 = g * NBUF
      for buf in range(NBUF):
        gather_wait(buf)
        transpose_scale(buf)
        scat_start(s0 + buf, buf)
      for buf in range(NBUF):
        scat_wait(s0 + buf, buf)
        gather_start(s0 + NBUF + buf, buf)
      return carry

    lax.fori_loop(0, n_groups - 1, group_body, 0)

    s0 = (n_groups - 1) * NBUF
    for buf in range(NBUF):
      gather_wait(buf)
      transpose_scale(buf)
      scat_start(s0 + buf, buf)
    for buf in range(NBUF):
      scat_wait(s0 + buf, buf)

  return emb_kernel


def kernel(x, weight):
  b, s = x.shape
  vocab, d = weight.shape
  w128 = weight.reshape(vocab // 2, 2 * d)
  x3 = x.T.reshape(s, NUM_WORKERS, b // NUM_WORKERS).astype(jnp.int32)
  out = _make_kernel(b, s, vocab, d)(x3, w128)
  return out.transpose(2, 0, 1)


# speedup vs baseline: 1.1532x; 1.1532x over previous
import functools
import math

import jax
import jax.numpy as jnp
from jax import lax
from jax.experimental import pallas as pl
from jax.experimental.pallas import tpu as pltpu
from jax.experimental.pallas import tpu_sc as plsc

D_MODEL = 64
SCALE = math.sqrt(D_MODEL)

NUM_CORES = 2
NUM_SUBCORES = 16
NUM_WORKERS = NUM_CORES * NUM_SUBCORES

LANES = 16
SPLIT = 104
NBUF = 4


def _make_kernel(b: int, s: int, vocab: int, d: int):
  assert b % (NUM_WORKERS * NBUF) == 0 and s % 8 == 0 and SPLIT % 8 == 0
  rows_per_w = b // NUM_WORKERS
  n_groups = rows_per_w // NBUF

  mesh = plsc.VectorSubcoreMesh(core_axis_name="c", subcore_axis_name="s")

  @functools.partial(
      pl.kernel,
      mesh=mesh,
      out_type=jax.ShapeDtypeStruct((b, s, d), jnp.float32),
      scratch_types=[
          pltpu.VMEM((rows_per_w, s), jnp.int32),
          [pltpu.VMEM((s, d), jnp.float32) for _ in range(NBUF)],
          [pltpu.SemaphoreType.DMA for _ in range(NBUF)],
          [pltpu.SemaphoreType.DMA for _ in range(NBUF)],
      ],
      compiler_params=pltpu.CompilerParams(use_tc_tiling_on_sc=False),
  )
  def emb_kernel(x_hbm, w_hbm, out_hbm, idx_v, rows, gat_sem, scat_sem):
    wid = lax.axis_index("s") * NUM_CORES + lax.axis_index("c")
    base = wid * rows_per_w
    pltpu.sync_copy(x_hbm.at[wid], idx_v)

    def gather_start(r, buf):
      pltpu.make_async_copy(
          w_hbm.at[idx_v.at[r, pl.ds(0, SPLIT)]],
          rows[buf].at[pl.ds(0, SPLIT)],
          gat_sem[buf],
      ).start()
      pltpu.make_async_copy(
          w_hbm.at[idx_v.at[r, pl.ds(SPLIT, s - SPLIT)]],
          rows[buf].at[pl.ds(SPLIT, s - SPLIT)],
          gat_sem[buf],
      ).start()

    def gather_wait(r, buf):
      pltpu.make_async_copy(
          w_hbm.at[idx_v.at[r, pl.ds(0, SPLIT)]],
          rows[buf].at[pl.ds(0, SPLIT)],
          gat_sem[buf],
      ).wait()
      pltpu.make_async_copy(
          w_hbm.at[idx_v.at[r, pl.ds(SPLIT, s - SPLIT)]],
          rows[buf].at[pl.ds(SPLIT, s - SPLIT)],
          gat_sem[buf],
      ).wait()

    def scat_start(r, buf):
      pltpu.make_async_copy(rows[buf], out_hbm.at[base + r], scat_sem[buf]).start()

    def scat_wait(r, buf):
      pltpu.make_async_copy(rows[buf], out_hbm.at[base + r], scat_sem[buf]).wait()

    def scale(buf):
      def tok_body(t, c2):
        for col in range(d // LANES):
          sl = pl.ds(col * LANES, LANES)
          rows[buf][t, sl] = rows[buf][t, sl] * SCALE
        return c2

      lax.fori_loop(0, s, tok_body, 0, unroll=4)

    for buf in range(NBUF):
      gather_start(buf, buf)

    def group_body(g, carry):
      r0 = g * NBUF
      for buf in range(NBUF):
        gather_wait(r0 + buf, buf)
        scale(buf)
        scat_start(r0 + buf, buf)
      for buf in range(NBUF):
        scat_wait(r0 + buf, buf)
        gather_start(r0 + NBUF + buf, buf)
      return carry

    lax.fori_loop(0, n_groups - 1, group_body, 0)

    r0 = (n_groups - 1) * NBUF
    for buf in range(NBUF):
      gather_wait(r0 + buf, buf)
      scale(buf)
      scat_start(r0 + buf, buf)
    for buf in range(NBUF):
      scat_wait(r0 + buf, buf)

  return emb_kernel


def kernel(x, weight):
  b, s = x.shape
  vocab, d = weight.shape
  x3 = x.reshape(NUM_WORKERS, b // NUM_WORKERS, s).astype(jnp.int32)
  return _make_kernel(b, s, vocab, d)(x3, weight)
